# baseline (speedup 1.0000x reference)
import jax
import jax.numpy as jnp
from jax.experimental import pallas as pl
from jax.experimental.pallas import tpu as pltpu

OUT_F = 4096
N_POS = 3072
LEVELS = 256
BATCH = 8
DT = 512


def _body(x_ref, pw_ref, vw_ref, out_ref, h_ref):
    @pl.when(pl.program_id(0) == 0)
    def _build():
        idx = jnp.clip(jnp.round(x_ref[...] * (LEVELS - 1)), 0, LEVELS - 1).astype(
            jnp.int32
        )
        lv = jax.lax.broadcasted_iota(jnp.int32, (LEVELS, N_POS), 0)
        for b in range(BATCH):
            h_ref[b * LEVELS : (b + 1) * LEVELS, :] = (
                lv == idx[b : b + 1, :]
            ).astype(jnp.bfloat16)

    pwb = pw_ref[...].astype(jnp.bfloat16)
    vw = vw_ref[...]
    ones = jnp.ones((1, LEVELS), jnp.float32)
    for b in range(BATCH):
        g = jax.lax.dot_general(
            h_ref[b * LEVELS : (b + 1) * LEVELS, :],
            pwb,
            (((1,), (0,)), ((), ())),
            preferred_element_type=jnp.float32,
        )
        ms = jax.lax.dot_general(
            ones, g * vw, (((1,), (0,)), ((), ())),
            preferred_element_type=jnp.float32,
        )
        out_ref[b : b + 1, :] = jnp.where(ms > 0, 1.0, -1.0)


def kernel(x, position_weight, value_weight):
    xf = x.reshape(BATCH, N_POS)
    return pl.pallas_call(
        _body,
        grid=(OUT_F // DT,),
        in_specs=[
            pl.BlockSpec((BATCH, N_POS), lambda j: (0, 0)),
            pl.BlockSpec((N_POS, DT), lambda j: (0, j)),
            pl.BlockSpec((LEVELS, DT), lambda j: (0, j)),
        ],
        out_specs=pl.BlockSpec((BATCH, DT), lambda j: (0, j)),
        out_shape=jax.ShapeDtypeStruct((BATCH, OUT_F), jnp.float32),
        scratch_shapes=[pltpu.VMEM((BATCH * LEVELS, N_POS), jnp.bfloat16)],
    )(xf, position_weight, value_weight)


# speedup vs baseline: 1.1777x; 1.1777x over previous
import jax
import jax.numpy as jnp
from jax.experimental import pallas as pl
from jax.experimental.pallas import tpu as pltpu

OUT_F = 4096
N_POS = 3072
LEVELS = 256
BATCH = 8
DT = 512


def _body(x_ref, pw_ref, vw_ref, out_ref, h_ref):
    @pl.when(pl.program_id(0) == 0)
    def _build():
        idx = jnp.clip(jnp.round(x_ref[...] * (LEVELS - 1)), 0, LEVELS - 1).astype(
            jnp.int32
        ).astype(jnp.bfloat16)
        lv = jax.lax.broadcasted_iota(jnp.int32, (LEVELS, N_POS), 0).astype(
            jnp.bfloat16
        )
        one = jnp.ones((LEVELS, N_POS), jnp.bfloat16)
        zero = jnp.zeros((LEVELS, N_POS), jnp.bfloat16)
        for b in range(BATCH):
            h_ref[b * LEVELS : (b + 1) * LEVELS, :] = jnp.where(
                lv == idx[b : b + 1, :], one, zero
            )

    pwb = pw_ref[...].astype(jnp.bfloat16)
    vw = vw_ref[...]
    for b in range(BATCH):
        g = jax.lax.dot_general(
            h_ref[b * LEVELS : (b + 1) * LEVELS, :],
            pwb,
            (((1,), (0,)), ((), ())),
            preferred_element_type=jnp.float32,
        )
        ms = jnp.sum(g * vw, axis=0, keepdims=True)
        out_ref[b : b + 1, :] = jnp.where(ms > 0, 1.0, -1.0)


def kernel(x, position_weight, value_weight):
    xf = x.reshape(BATCH, N_POS)
    return pl.pallas_call(
        _body,
        grid=(OUT_F // DT,),
        in_specs=[
            pl.BlockSpec((BATCH, N_POS), lambda j: (0, 0)),
            pl.BlockSpec((N_POS, DT), lambda j: (0, j)),
            pl.BlockSpec((LEVELS, DT), lambda j: (0, j)),
        ],
        out_specs=pl.BlockSpec((BATCH, DT), lambda j: (0, j)),
        out_shape=jax.ShapeDtypeStruct((BATCH, OUT_F), jnp.float32),
        scratch_shapes=[pltpu.VMEM((BATCH * LEVELS, N_POS), jnp.bfloat16)],
    )(xf, position_weight, value_weight)
